# Initial kernel scaffold; baseline (speedup 1.0000x reference)
import jax
import jax.numpy as jnp
from jax.experimental import pallas as pl


def kernel(cls_scores_0, cls_scores_1, cls_scores_2, cls_scores_3, cls_scores_4, bbox_preds_0, bbox_preds_1, bbox_preds_2, bbox_preds_3, bbox_preds_4, centernesses_0, centernesses_1, centernesses_2, centernesses_3, centernesses_4):
    raise NotImplementedError("write your pallas kernel here")


# speedup vs baseline: 1.1439x; 1.1439x over previous
import jax
import jax.numpy as jnp
from jax.experimental import pallas as pl
from jax.experimental.pallas import tpu as pltpu

_STRIDES = [8, 16, 32, 64, 128]
_THRESHOLD = 0.99
_HWS = [64, 32, 16, 8, 4]
_NLOC = [hw * hw for hw in _HWS]
_OFFS = [0, 4096, 5120, 5376, 5440]
_TOT = 5456
_C = 80
_OUTC = 87


def _placement(rows, col0):
    r = jax.lax.broadcasted_iota(jnp.int32, (rows, _OUTC), 0)
    c = jax.lax.broadcasted_iota(jnp.int32, (rows, _OUTC), 1)
    return (c == r + col0).astype(jnp.float32)


def _body(c0, c1, c2, c3, c4, b0, b1, b2, b3, b4, t0, t1, t2, t3, t4, out_ref):
    cls_refs = [c0, c1, c2, c3, c4]
    bbox_refs = [b0, b1, b2, b3, b4]
    ctr_refs = [t0, t1, t2, t3, t4]
    dn = (((0,), (0,)), ((), ()))
    for l in range(5):
        m = _NLOC[l]
        hw = _HWS[l]
        x = cls_refs[l][0]
        bb = bbox_refs[l][0]
        ct = ctr_refs[l][0]
        maxs = jnp.max(x, axis=0, keepdims=True)
        mask = (maxs > _THRESHOLD).astype(jnp.float32)
        xm = x * mask
        bbm = bb * mask
        ctm = ct * mask
        t = jax.lax.dot_general(xm, _placement(_C, 2), dn,
                                preferred_element_type=jnp.float32)
        t += jax.lax.dot_general(bbm, _placement(4, 82), dn,
                                 preferred_element_type=jnp.float32)
        t += jax.lax.dot_general(ctm, _placement(1, 86), dn,
                                 preferred_element_type=jnp.float32)
        tm = jax.lax.dot_general(mask, jnp.ones((1, _OUTC), jnp.float32), dn,
                                 preferred_element_type=jnp.float32)
        r = jax.lax.broadcasted_iota(jnp.int32, (m, _OUTC), 0)
        c = jax.lax.broadcasted_iota(jnp.int32, (m, _OUTC), 1)
        xs = ((r % hw) * _STRIDES[l]).astype(jnp.float32)
        ys = ((r // hw) * _STRIDES[l]).astype(jnp.float32)
        coords = jnp.where(c == 0, xs, jnp.where(c == 1, ys, 0.0))
        out_ref[0, _OFFS[l]:_OFFS[l] + m, :] = t + coords * tm


def kernel(cls_scores_0, cls_scores_1, cls_scores_2, cls_scores_3, cls_scores_4,
           bbox_preds_0, bbox_preds_1, bbox_preds_2, bbox_preds_3, bbox_preds_4,
           centernesses_0, centernesses_1, centernesses_2, centernesses_3,
           centernesses_4):
    n = cls_scores_0.shape[0]
    cls_l = [cls_scores_0, cls_scores_1, cls_scores_2, cls_scores_3, cls_scores_4]
    bbox_l = [bbox_preds_0, bbox_preds_1, bbox_preds_2, bbox_preds_3, bbox_preds_4]
    ctr_l = [centernesses_0, centernesses_1, centernesses_2, centernesses_3,
             centernesses_4]
    args = []
    specs = []
    for lst, ch in ((cls_l, _C), (bbox_l, 4), (ctr_l, 1)):
        for l in range(5):
            args.append(lst[l].reshape(n, ch, _NLOC[l]))
            specs.append(pl.BlockSpec((1, ch, _NLOC[l]), lambda i: (i, 0, 0)))
    return pl.pallas_call(
        _body,
        grid=(n,),
        in_specs=specs,
        out_specs=pl.BlockSpec((1, _TOT, _OUTC), lambda i: (i, 0, 0)),
        out_shape=jax.ShapeDtypeStruct((n, _TOT, _OUTC), jnp.float32),
    )(*args)
